# baseline (speedup 1.0000x reference)
import functools

import jax
import jax.numpy as jnp
from jax import lax
from jax.experimental import pallas as pl
from jax.experimental.pallas import tpu as pltpu
from jax.experimental.pallas import tpu_sc as plsc

NUM_CORES = 2
NUM_SUBCORES = 16
NUM_WORKERS = NUM_CORES * NUM_SUBCORES
CHUNK = 128


def _make_gather(n_rows: int, d: int, n_per_w: int):
    n_chunks = n_per_w // CHUNK
    mesh = plsc.VectorSubcoreMesh(
        core_axis_name="c", subcore_axis_name="s",
        num_cores=NUM_CORES, num_subcores=NUM_SUBCORES)

    @functools.partial(
        pl.kernel,
        out_type=jax.ShapeDtypeStruct((n_rows, d), jnp.float32),
        mesh=mesh,
        scratch_types=[
            pltpu.VMEM((n_per_w,), jnp.int32),
            pltpu.VMEM((n_per_w, d), jnp.float32),
            pltpu.SemaphoreType.DMA,
        ],
        compiler_params=pltpu.CompilerParams(use_tc_tiling_on_sc=False),
    )
    def gather(table_hbm, idx_hbm, out_hbm, idx_v, rows_v, sem):
        wid = lax.axis_index("s") * NUM_CORES + lax.axis_index("c")
        base = wid * n_per_w
        pltpu.sync_copy(idx_hbm.at[pl.ds(base, n_per_w)], idx_v)

        @pl.loop(0, n_chunks)
        def _(j):
            off = j * CHUNK
            pltpu.async_copy(
                table_hbm.at[idx_v.at[pl.ds(off, CHUNK)]],
                rows_v.at[pl.ds(off, CHUNK)],
                sem,
            ).wait()

        pltpu.sync_copy(rows_v, out_hbm.at[pl.ds(base, n_per_w)])

    return gather


def kernel(x, tables):
    batch, num_fields = x.shape
    num_emb, d = tables.shape[1], tables.shape[2]
    x = jnp.where(x < 0, num_emb - 1, x)
    offs = (jnp.arange(num_fields, dtype=jnp.int32) * num_emb)[None, :]
    gidx = (x + offs).reshape(-1)
    table_flat = tables.reshape(num_fields * num_emb, d)

    n_rows = batch * num_fields
    n_per_w = n_rows // NUM_WORKERS
    out = _make_gather(n_rows, d, n_per_w)(table_flat, gidx)
    return out.reshape(batch, num_fields * d)


# speedup vs baseline: 1.0083x; 1.0083x over previous
import functools

import jax
import jax.numpy as jnp
from jax import lax
from jax.experimental import pallas as pl
from jax.experimental.pallas import tpu as pltpu
from jax.experimental.pallas import tpu_sc as plsc

NUM_CORES = 2
NUM_SUBCORES = 16
NUM_WORKERS = NUM_CORES * NUM_SUBCORES
CHUNK = 128
DEPTH = 8


def _make_gather(n_rows: int, d: int, n_per_w: int):
    n_chunks = n_per_w // CHUNK
    mesh = plsc.VectorSubcoreMesh(
        core_axis_name="c", subcore_axis_name="s",
        num_cores=NUM_CORES, num_subcores=NUM_SUBCORES)

    @functools.partial(
        pl.kernel,
        out_type=jax.ShapeDtypeStruct((n_rows, d), jnp.float32),
        mesh=mesh,
        scratch_types=[
            pltpu.VMEM((n_per_w,), jnp.int32),
            pltpu.VMEM((n_per_w, d), jnp.float32),
            pltpu.SemaphoreType.DMA,
        ],
        compiler_params=pltpu.CompilerParams(use_tc_tiling_on_sc=False),
    )
    def gather(table_hbm, idx_hbm, out_hbm, idx_v, rows_v, sem):
        wid = lax.axis_index("s") * NUM_CORES + lax.axis_index("c")
        base = wid * n_per_w
        pltpu.sync_copy(idx_hbm.at[pl.ds(base, n_per_w)], idx_v)

        @pl.loop(0, n_chunks // DEPTH)
        def _(g):
            goff = g * (DEPTH * CHUNK)
            descs = []
            for b in range(DEPTH):
                off = goff + b * CHUNK
                descs.append(pltpu.async_copy(
                    table_hbm.at[idx_v.at[pl.ds(off, CHUNK)]],
                    rows_v.at[pl.ds(off, CHUNK)],
                    sem,
                ))
            for desc in descs:
                desc.wait()

        pltpu.sync_copy(rows_v, out_hbm.at[pl.ds(base, n_per_w)])

    return gather


def kernel(x, tables):
    batch, num_fields = x.shape
    num_emb, d = tables.shape[1], tables.shape[2]
    x = jnp.where(x < 0, num_emb - 1, x)
    offs = (jnp.arange(num_fields, dtype=jnp.int32) * num_emb)[None, :]
    gidx = (x + offs).reshape(-1)
    table_flat = tables.reshape(num_fields * num_emb, d)

    n_rows = batch * num_fields
    n_per_w = n_rows // NUM_WORKERS
    out = _make_gather(n_rows, d, n_per_w)(table_flat, gidx)
    return out.reshape(batch, num_fields * d)
